# baseline (speedup 1.0000x reference)
import functools

import jax
import jax.numpy as jnp
from jax.experimental import pallas as pl
from jax.experimental.pallas import tpu as pltpu

D = 128
H = 128
A = 16
G = 64
NEG = -1e30


def _blk(n, pref):
    return pref if n % pref == 0 else n


def _k1_body(x_ref, fb_ref, wx_ref, scal_ref, big_ref):
    y = jnp.dot(x_ref[...], wx_ref[...], preferred_element_type=jnp.float32)
    bn = y.shape[0]
    scal_ref[...] = jnp.concatenate(
        [y[:, :4], fb_ref[...], jnp.zeros((bn, 3), jnp.float32)], axis=1)
    big_ref[...] = y[:, 4:]


def _k1(x, fbatch, wx, n):
    bn = _blk(n, 2000)
    grid = n // bn
    return pl.pallas_call(
        _k1_body,
        grid=(grid,),
        in_specs=[
            pl.BlockSpec((bn, D), lambda i: (i, 0)),
            pl.BlockSpec((bn, 1), lambda i: (i, 0)),
            pl.BlockSpec((D, 516), lambda i: (0, 0)),
        ],
        out_specs=[
            pl.BlockSpec((bn, 8), lambda i: (i, 0)),
            pl.BlockSpec((bn, 512), lambda i: (i, 0)),
        ],
        out_shape=[
            jax.ShapeDtypeStruct((n, 8), jnp.float32),
            jax.ShapeDtypeStruct((n, 512), jnp.float32),
        ],
        compiler_params=pltpu.CompilerParams(
            dimension_semantics=("arbitrary",)),
    )(x, fbatch, wx)


def _p1_body(rol_ref, col_ref, ea_ref, ef_ref, scal_ref, wattr_ref, wf_ref,
             lg_ref, m_ref, sgr, sgc):
    @pl.when(pl.program_id(0) == 0)
    def _():
        m_ref[...] = jnp.full(m_ref.shape, NEG, jnp.float32)

    be = ea_ref.shape[0]
    part = (jnp.dot(ea_ref[...], wattr_ref[...],
                    preferred_element_type=jnp.float32) +
            jnp.dot(ef_ref[...], wf_ref[...],
                    preferred_element_type=jnp.float32))

    def gather(i, c):
        r = rol_ref[0, 0, i]
        cc = col_ref[0, 0, i]
        sgr[pl.ds(i, 1), :] = scal_ref[pl.ds(r, 1), :]
        sgc[pl.ds(i, 1), :] = scal_ref[pl.ds(cc, 1), :]
        return c
    jax.lax.fori_loop(0, be, gather, 0)

    lg_ref[...] = part + sgr[:, 0:1] + sgc[:, 1:2]

    def smax(i, c):
        cc = col_ref[0, 0, i]
        lv = lg_ref[pl.ds(i, 1), :]
        m_ref[pl.ds(cc, 1), :] = jnp.maximum(m_ref[pl.ds(cc, 1), :], lv)
        return c
    jax.lax.fori_loop(0, be, smax, 0)


def _p1(rol2, col2, ea, ef, scal, wattr, wf, n, e):
    be = _blk(e, 2000)
    grid = e // be
    return pl.pallas_call(
        _p1_body,
        grid=(grid,),
        in_specs=[
            pl.BlockSpec((1, 1, be), lambda i: (i, 0, 0), memory_space=pltpu.SMEM),
            pl.BlockSpec((1, 1, be), lambda i: (i, 0, 0), memory_space=pltpu.SMEM),
            pl.BlockSpec((be, A), lambda i: (i, 0)),
            pl.BlockSpec((be, H), lambda i: (i, 0)),
            pl.BlockSpec((n, 8), lambda i: (0, 0)),
            pl.BlockSpec((A, 1), lambda i: (0, 0)),
            pl.BlockSpec((H, 1), lambda i: (0, 0)),
        ],
        out_specs=[
            pl.BlockSpec((be, 1), lambda i: (i, 0)),
            pl.BlockSpec((n, 1), lambda i: (0, 0)),
        ],
        out_shape=[
            jax.ShapeDtypeStruct((e, 1), jnp.float32),
            jax.ShapeDtypeStruct((n, 1), jnp.float32),
        ],
        scratch_shapes=[
            pltpu.VMEM((be, 8), jnp.float32),
            pltpu.VMEM((be, 8), jnp.float32),
        ],
        compiler_params=pltpu.CompilerParams(
            dimension_semantics=("arbitrary",)),
    )(rol2, col2, ea, ef, scal, wattr, wf)


def _p2_body(col_ref, lg_ref, ef_ref, m_ref, s_ref, u_ref):
    @pl.when(pl.program_id(0) == 0)
    def _():
        s_ref[...] = jnp.zeros(s_ref.shape, jnp.float32)
        u_ref[...] = jnp.zeros(u_ref.shape, jnp.float32)

    be = lg_ref.shape[0]

    def body(i, c):
        cc = col_ref[0, 0, i]
        ev = jnp.exp(lg_ref[pl.ds(i, 1), :] - m_ref[pl.ds(cc, 1), :])
        s_ref[pl.ds(cc, 1), :] = s_ref[pl.ds(cc, 1), :] + ev
        u_ref[pl.ds(cc, 1), :] = (u_ref[pl.ds(cc, 1), :] +
                                  ev * ef_ref[pl.ds(i, 1), :])
        return c
    jax.lax.fori_loop(0, be, body, 0)


def _p2(col2, lg, ef, m, n, e):
    be = _blk(e, 2000)
    grid = e // be
    return pl.pallas_call(
        _p2_body,
        grid=(grid,),
        in_specs=[
            pl.BlockSpec((1, 1, be), lambda i: (i, 0, 0), memory_space=pltpu.SMEM),
            pl.BlockSpec((be, 1), lambda i: (i, 0)),
            pl.BlockSpec((be, H), lambda i: (i, 0)),
            pl.BlockSpec((n, 1), lambda i: (0, 0)),
        ],
        out_specs=[
            pl.BlockSpec((n, 1), lambda i: (0, 0)),
            pl.BlockSpec((n, H), lambda i: (0, 0)),
        ],
        out_shape=[
            jax.ShapeDtypeStruct((n, 1), jnp.float32),
            jax.ShapeDtypeStruct((n, H), jnp.float32),
        ],
        compiler_params=pltpu.CompilerParams(
            dimension_semantics=("arbitrary",)),
    )(col2, lg, ef, m)


def _k3_body(u_ref, s_ref, big_ref, scal_ref, e1b_ref, e1d_ref,
             r4_ref, c4_ref):
    bn = u_ref.shape[0]
    nf = u_ref[...] / (s_ref[...] + 1e-16)
    t_r = big_ref[:, 0:128] + jnp.dot(nf, e1b_ref[...],
                                      preferred_element_type=jnp.float32)
    t_c = big_ref[:, 128:256] + jnp.dot(nf, e1d_ref[...],
                                        preferred_element_type=jnp.float32)
    z126 = jnp.zeros((bn, 126), jnp.float32)
    z127 = jnp.zeros((bn, 127), jnp.float32)
    r4_ref[...] = jnp.concatenate(
        [t_r, big_ref[:, 256:384],
         scal_ref[:, 2:3], scal_ref[:, 4:5], z126], axis=1)
    c4_ref[...] = jnp.concatenate(
        [t_c, big_ref[:, 384:512], scal_ref[:, 3:4], z127], axis=1)


def _k3(u, s, big, scal, e1b, e1d, n):
    bn = _blk(n, 2000)
    grid = n // bn
    return pl.pallas_call(
        _k3_body,
        grid=(grid,),
        in_specs=[
            pl.BlockSpec((bn, H), lambda i: (i, 0)),
            pl.BlockSpec((bn, 1), lambda i: (i, 0)),
            pl.BlockSpec((bn, 512), lambda i: (i, 0)),
            pl.BlockSpec((bn, 8), lambda i: (i, 0)),
            pl.BlockSpec((H, H), lambda i: (0, 0)),
            pl.BlockSpec((H, H), lambda i: (0, 0)),
        ],
        out_specs=[
            pl.BlockSpec((bn, 384), lambda i: (i, 0)),
            pl.BlockSpec((bn, 384), lambda i: (i, 0)),
        ],
        out_shape=[
            jax.ShapeDtypeStruct((n, 384), jnp.float32),
            jax.ShapeDtypeStruct((n, 384), jnp.float32),
        ],
        compiler_params=pltpu.CompilerParams(
            dimension_semantics=("arbitrary",)),
    )(u, s, big, scal, e1b, e1d)


def _p4_body(rol_ref, col_ref, ea_ref, ef_ref, r4_ref, c4_ref,
             e1e_ref, we2_ref, be1_ref, be2_ref,
             uattr_ref, uf_ref, uc_ref, bupd_ref, rf_ref, rattr_ref,
             newf_ref, ro_ref, gseg_ref, gm_ref, gr, gc):
    @pl.when(pl.program_id(0) == 0)
    def _():
        gm_ref[...] = jnp.full(gm_ref.shape, NEG, jnp.float32)

    be = ea_ref.shape[0]

    def gather(i, c):
        r = rol_ref[0, 0, i]
        cc = col_ref[0, 0, i]
        gr[pl.ds(i, 1), :] = r4_ref[pl.ds(r, 1), :]
        gc[pl.ds(i, 1), :] = c4_ref[pl.ds(cc, 1), :]
        return c
    jax.lax.fori_loop(0, be, gather, 0)

    ea = ea_ref[...]
    ef = ef_ref[...]
    emb_pre = (jnp.dot(ea, e1e_ref[...], preferred_element_type=jnp.float32)
               + be1_ref[...] + gr[:, 0:128] + gc[:, 0:128])
    emb = (jnp.dot(jnp.maximum(emb_pre, 0.0), we2_ref[...],
                   preferred_element_type=jnp.float32) + be2_ref[...])
    gate_pre = (jnp.dot(ea, uattr_ref[...], preferred_element_type=jnp.float32)
                + jnp.dot(ef, uf_ref[...], preferred_element_type=jnp.float32)
                + jnp.dot(emb, uc_ref[...], preferred_element_type=jnp.float32)
                + bupd_ref[...] + gr[:, 128:256] + gc[:, 128:256])
    gate = jax.nn.sigmoid(gate_pre)
    nfe = gate * emb + (1.0 - gate) * ef
    newf_ref[...] = nfe

    ro = (jnp.dot(nfe, rf_ref[...], preferred_element_type=jnp.float32)
          + jnp.dot(ea, rattr_ref[...], preferred_element_type=jnp.float32)
          + gr[:, 256:257] + gc[:, 256:257])
    ro_ref[...] = ro
    gseg = gr[:, 257:258]
    gseg_ref[...] = gseg

    iota = jax.lax.broadcasted_iota(jnp.int32, (be, G), 1)
    mask = gseg.astype(jnp.int32) == iota
    cand = jnp.where(mask, ro, NEG)
    gm_ref[...] = jnp.maximum(gm_ref[...],
                              jnp.max(cand, axis=0, keepdims=True))


def _p4(rol2, col2, ea, ef, r4, c4, e1e, we2, be1, be2,
        uattr, uf, uc, bupd, rf, rattr, n, e):
    be = _blk(e, 2000)
    grid = e // be
    full = lambda i: (0, 0)
    return pl.pallas_call(
        _p4_body,
        grid=(grid,),
        in_specs=[
            pl.BlockSpec((1, 1, be), lambda i: (i, 0, 0), memory_space=pltpu.SMEM),
            pl.BlockSpec((1, 1, be), lambda i: (i, 0, 0), memory_space=pltpu.SMEM),
            pl.BlockSpec((be, A), lambda i: (i, 0)),
            pl.BlockSpec((be, H), lambda i: (i, 0)),
            pl.BlockSpec((n, 384), full),
            pl.BlockSpec((n, 384), full),
            pl.BlockSpec((A, H), full),
            pl.BlockSpec((H, H), full),
            pl.BlockSpec((1, H), full),
            pl.BlockSpec((1, H), full),
            pl.BlockSpec((A, H), full),
            pl.BlockSpec((H, H), full),
            pl.BlockSpec((H, H), full),
            pl.BlockSpec((1, H), full),
            pl.BlockSpec((H, 1), full),
            pl.BlockSpec((A, 1), full),
        ],
        out_specs=[
            pl.BlockSpec((be, H), lambda i: (i, 0)),
            pl.BlockSpec((be, 1), lambda i: (i, 0)),
            pl.BlockSpec((be, 1), lambda i: (i, 0)),
            pl.BlockSpec((1, G), full),
        ],
        out_shape=[
            jax.ShapeDtypeStruct((e, H), jnp.float32),
            jax.ShapeDtypeStruct((e, 1), jnp.float32),
            jax.ShapeDtypeStruct((e, 1), jnp.float32),
            jax.ShapeDtypeStruct((1, G), jnp.float32),
        ],
        scratch_shapes=[
            pltpu.VMEM((be, 384), jnp.float32),
            pltpu.VMEM((be, 384), jnp.float32),
        ],
        compiler_params=pltpu.CompilerParams(
            dimension_semantics=("arbitrary",)),
    )(rol2, col2, ea, ef, r4, c4, e1e, we2, be1, be2,
      uattr, uf, uc, bupd, rf, rattr)


def _p5_body(newf_ref, ro_ref, gseg_ref, gm_ref, ws_ref, bs_ref,
             conf_ref, gu, gs):
    @pl.when(pl.program_id(0) == 0)
    def _():
        gu[...] = jnp.zeros(gu.shape, jnp.float32)
        gs[...] = jnp.zeros(gs.shape, jnp.float32)

    be = ro_ref.shape[0]
    iota = jax.lax.broadcasted_iota(jnp.int32, (be, G), 1)
    mask = gseg_ref[...].astype(jnp.int32) == iota
    maskf = mask.astype(jnp.float32)
    gmrow = jnp.sum(jnp.where(mask, gm_ref[...], 0.0), axis=1, keepdims=True)
    ev = jnp.exp(ro_ref[...] - gmrow)
    dn = (((0,), (0,)), ((), ()))
    gs[...] = gs[...] + jax.lax.dot_general(
        maskf, ev, dn, preferred_element_type=jnp.float32)
    gu[...] = gu[...] + jax.lax.dot_general(
        maskf, ev * newf_ref[...], dn, preferred_element_type=jnp.float32)

    gf = gu[...] / (gs[...] + 1e-16)
    conf_ref[...] = jax.nn.sigmoid(
        jnp.dot(gf, ws_ref[...], preferred_element_type=jnp.float32)
        + bs_ref[...])


def _p5(newf, ro, gseg, gm, ws, bs, e):
    be = _blk(e, 2000)
    grid = e // be
    full = lambda i: (0, 0)
    return pl.pallas_call(
        _p5_body,
        grid=(grid,),
        in_specs=[
            pl.BlockSpec((be, H), lambda i: (i, 0)),
            pl.BlockSpec((be, 1), lambda i: (i, 0)),
            pl.BlockSpec((be, 1), lambda i: (i, 0)),
            pl.BlockSpec((1, G), full),
            pl.BlockSpec((H, 1), full),
            pl.BlockSpec((1, 1), full),
        ],
        out_specs=[pl.BlockSpec((G, 1), full)],
        out_shape=[jax.ShapeDtypeStruct((G, 1), jnp.float32)],
        scratch_shapes=[
            pltpu.VMEM((G, H), jnp.float32),
            pltpu.VMEM((G, 1), jnp.float32),
        ],
        compiler_params=pltpu.CompilerParams(
            dimension_semantics=("arbitrary",)),
    )(newf, ro, gseg, gm, ws, bs)


def kernel(x, hidden_edge_feat, edge_index, edge_attr, batch, num_graphs,
           W_agg_att, b_agg_att, W_e1, b_e1, W_e2, b_e2,
           W_upd, b_upd, W_ro_att, b_ro_att, W_s, b_s):
    n = x.shape[0]
    e = edge_index.shape[1]

    wa_r, wa_c = W_agg_att[0:D], W_agg_att[D:2 * D]
    wa_attr, wa_f = W_agg_att[2 * D:2 * D + A], W_agg_att[2 * D + A:]
    e1a, e1b = W_e1[0:D], W_e1[D:2 * D]
    e1c, e1d, e1e = W_e1[2 * D:3 * D], W_e1[3 * D:4 * D], W_e1[4 * D:]
    u_a, u_b = W_upd[0:D], W_upd[D:2 * D]
    u_attr = W_upd[2 * D:2 * D + A]
    u_f = W_upd[2 * D + A:2 * D + A + H]
    u_c = W_upd[2 * D + A + H:]
    r_a, r_b = W_ro_att[0:D], W_ro_att[D:2 * D]
    r_f, r_attr = W_ro_att[2 * D:2 * D + H], W_ro_att[2 * D + H:]

    wx = jnp.concatenate([wa_r, wa_c, r_a, r_b, e1a, e1c, u_a, u_b], axis=1)
    fbatch = batch.astype(jnp.float32).reshape(n, 1)
    bei = _blk(e, 2000)
    rol2 = edge_index[0].reshape(e // bei, 1, bei)
    col2 = edge_index[1].reshape(e // bei, 1, bei)
    be1 = (b_e1.reshape(1, H) + 0.0)
    be2 = b_e2.reshape(1, H)
    bupd = b_upd.reshape(1, H)
    bs = b_s.reshape(1, 1)
    del b_agg_att, b_ro_att, num_graphs

    scal, big = _k1(x, fbatch, wx, n)
    lg, m = _p1(rol2, col2, edge_attr, hidden_edge_feat, scal,
                wa_attr, wa_f, n, e)
    s, u = _p2(col2, lg, hidden_edge_feat, m, n, e)
    r4, c4 = _k3(u, s, big, scal, e1b, e1d, n)
    newf, ro, gseg, gm = _p4(rol2, col2, edge_attr, hidden_edge_feat,
                             r4, c4, e1e, W_e2, be1, be2,
                             u_attr, u_f, u_c, bupd, r_f, r_attr, n, e)
    (conf,) = _p5(newf, ro, gseg, gm, W_s, bs, e)
    return (newf, conf)


# speedup vs baseline: 4.2162x; 4.2162x over previous
import functools

import jax
import jax.numpy as jnp
from jax.experimental import pallas as pl
from jax.experimental.pallas import tpu as pltpu

D = 128
H = 128
A = 16
G = 64
NEG = -1e30


def _blk(n, pref):
    return pref if n % pref == 0 else n


def _k1_body(x_ref, fb_ref, wx_ref, scal_ref, big_ref):
    y = jnp.dot(x_ref[...], wx_ref[...], preferred_element_type=jnp.float32)
    bn = y.shape[0]
    scal_ref[...] = jnp.concatenate(
        [y[:, :4], fb_ref[...], jnp.zeros((bn, 3), jnp.float32)], axis=1)
    big_ref[...] = y[:, 4:]


def _k1(x, fbatch, wx, n):
    bn = _blk(n, 2000)
    grid = n // bn
    return pl.pallas_call(
        _k1_body,
        grid=(grid,),
        in_specs=[
            pl.BlockSpec((bn, D), lambda i: (i, 0)),
            pl.BlockSpec((bn, 1), lambda i: (i, 0)),
            pl.BlockSpec((D, 516), lambda i: (0, 0)),
        ],
        out_specs=[
            pl.BlockSpec((bn, 8), lambda i: (i, 0)),
            pl.BlockSpec((bn, 512), lambda i: (i, 0)),
        ],
        out_shape=[
            jax.ShapeDtypeStruct((n, 8), jnp.float32),
            jax.ShapeDtypeStruct((n, 512), jnp.float32),
        ],
        compiler_params=pltpu.CompilerParams(
            dimension_semantics=("arbitrary",)),
    )(x, fbatch, wx)


def _p1_body(rol_ref, col_ref, ea_ref, ef_ref, scal_ref, wattr_ref, wf_ref,
             lg_ref, m_ref, sgr, sgc):
    @pl.when(pl.program_id(0) == 0)
    def _():
        m_ref[...] = jnp.full(m_ref.shape, NEG, jnp.float32)

    be = ea_ref.shape[0]
    part = (jnp.dot(ea_ref[...], wattr_ref[...],
                    preferred_element_type=jnp.float32) +
            jnp.dot(ef_ref[...], wf_ref[...],
                    preferred_element_type=jnp.float32))

    def gather(i, c):
        r = rol_ref[0, 0, i]
        cc = col_ref[0, 0, i]
        sgr[pl.ds(i, 1), :] = scal_ref[pl.ds(r, 1), :]
        sgc[pl.ds(i, 1), :] = scal_ref[pl.ds(cc, 1), :]
        return c
    jax.lax.fori_loop(0, be, gather, 0, unroll=8)

    lg_ref[...] = part + sgr[:, 0:1] + sgc[:, 1:2]

    def smax(i, c):
        cc = col_ref[0, 0, i]
        lv = lg_ref[pl.ds(i, 1), :]
        m_ref[pl.ds(cc, 1), :] = jnp.maximum(m_ref[pl.ds(cc, 1), :], lv)
        return c
    jax.lax.fori_loop(0, be, smax, 0, unroll=8)


def _p1(rol2, col2, ea, ef, scal, wattr, wf, n, e):
    be = _blk(e, 2000)
    grid = e // be
    return pl.pallas_call(
        _p1_body,
        grid=(grid,),
        in_specs=[
            pl.BlockSpec((1, 1, be), lambda i: (i, 0, 0), memory_space=pltpu.SMEM),
            pl.BlockSpec((1, 1, be), lambda i: (i, 0, 0), memory_space=pltpu.SMEM),
            pl.BlockSpec((be, A), lambda i: (i, 0)),
            pl.BlockSpec((be, H), lambda i: (i, 0)),
            pl.BlockSpec((n, 8), lambda i: (0, 0)),
            pl.BlockSpec((A, 1), lambda i: (0, 0)),
            pl.BlockSpec((H, 1), lambda i: (0, 0)),
        ],
        out_specs=[
            pl.BlockSpec((be, 1), lambda i: (i, 0)),
            pl.BlockSpec((n, 1), lambda i: (0, 0)),
        ],
        out_shape=[
            jax.ShapeDtypeStruct((e, 1), jnp.float32),
            jax.ShapeDtypeStruct((n, 1), jnp.float32),
        ],
        scratch_shapes=[
            pltpu.VMEM((be, 8), jnp.float32),
            pltpu.VMEM((be, 8), jnp.float32),
        ],
        compiler_params=pltpu.CompilerParams(
            dimension_semantics=("arbitrary",)),
    )(rol2, col2, ea, ef, scal, wattr, wf)


def _p2_body(col_ref, lg_ref, ef_ref, m_ref, s_ref, u_ref):
    @pl.when(pl.program_id(0) == 0)
    def _():
        s_ref[...] = jnp.zeros(s_ref.shape, jnp.float32)
        u_ref[...] = jnp.zeros(u_ref.shape, jnp.float32)

    be = lg_ref.shape[0]

    def body(i, c):
        cc = col_ref[0, 0, i]
        ev = jnp.exp(lg_ref[pl.ds(i, 1), :] - m_ref[pl.ds(cc, 1), :])
        s_ref[pl.ds(cc, 1), :] = s_ref[pl.ds(cc, 1), :] + ev
        u_ref[pl.ds(cc, 1), :] = (u_ref[pl.ds(cc, 1), :] +
                                  ev * ef_ref[pl.ds(i, 1), :])
        return c
    jax.lax.fori_loop(0, be, body, 0, unroll=8)


def _p2(col2, lg, ef, m, n, e):
    be = _blk(e, 2000)
    grid = e // be
    return pl.pallas_call(
        _p2_body,
        grid=(grid,),
        in_specs=[
            pl.BlockSpec((1, 1, be), lambda i: (i, 0, 0), memory_space=pltpu.SMEM),
            pl.BlockSpec((be, 1), lambda i: (i, 0)),
            pl.BlockSpec((be, H), lambda i: (i, 0)),
            pl.BlockSpec((n, 1), lambda i: (0, 0)),
        ],
        out_specs=[
            pl.BlockSpec((n, 1), lambda i: (0, 0)),
            pl.BlockSpec((n, H), lambda i: (0, 0)),
        ],
        out_shape=[
            jax.ShapeDtypeStruct((n, 1), jnp.float32),
            jax.ShapeDtypeStruct((n, H), jnp.float32),
        ],
        compiler_params=pltpu.CompilerParams(
            dimension_semantics=("arbitrary",)),
    )(col2, lg, ef, m)


def _k3_body(u_ref, s_ref, big_ref, scal_ref, e1b_ref, e1d_ref,
             r4_ref, c4_ref):
    bn = u_ref.shape[0]
    nf = u_ref[...] / (s_ref[...] + 1e-16)
    t_r = big_ref[:, 0:128] + jnp.dot(nf, e1b_ref[...],
                                      preferred_element_type=jnp.float32)
    t_c = big_ref[:, 128:256] + jnp.dot(nf, e1d_ref[...],
                                        preferred_element_type=jnp.float32)
    z126 = jnp.zeros((bn, 126), jnp.float32)
    z127 = jnp.zeros((bn, 127), jnp.float32)
    r4_ref[...] = jnp.concatenate(
        [t_r, big_ref[:, 256:384],
         scal_ref[:, 2:3], scal_ref[:, 4:5], z126], axis=1)
    c4_ref[...] = jnp.concatenate(
        [t_c, big_ref[:, 384:512], scal_ref[:, 3:4], z127], axis=1)


def _k3(u, s, big, scal, e1b, e1d, n):
    bn = _blk(n, 2000)
    grid = n // bn
    return pl.pallas_call(
        _k3_body,
        grid=(grid,),
        in_specs=[
            pl.BlockSpec((bn, H), lambda i: (i, 0)),
            pl.BlockSpec((bn, 1), lambda i: (i, 0)),
            pl.BlockSpec((bn, 512), lambda i: (i, 0)),
            pl.BlockSpec((bn, 8), lambda i: (i, 0)),
            pl.BlockSpec((H, H), lambda i: (0, 0)),
            pl.BlockSpec((H, H), lambda i: (0, 0)),
        ],
        out_specs=[
            pl.BlockSpec((bn, 384), lambda i: (i, 0)),
            pl.BlockSpec((bn, 384), lambda i: (i, 0)),
        ],
        out_shape=[
            jax.ShapeDtypeStruct((n, 384), jnp.float32),
            jax.ShapeDtypeStruct((n, 384), jnp.float32),
        ],
        compiler_params=pltpu.CompilerParams(
            dimension_semantics=("arbitrary",)),
    )(u, s, big, scal, e1b, e1d)


def _p4_body(rol_ref, col_ref, ea_ref, ef_ref, r4_ref, c4_ref,
             e1e_ref, we2_ref, be1_ref, be2_ref,
             uattr_ref, uf_ref, uc_ref, bupd_ref, rf_ref, rattr_ref,
             newf_ref, ro_ref, gseg_ref, gm_ref, gr, gc):
    @pl.when(pl.program_id(0) == 0)
    def _():
        gm_ref[...] = jnp.full(gm_ref.shape, NEG, jnp.float32)

    be = ea_ref.shape[0]

    def gather(i, c):
        r = rol_ref[0, 0, i]
        cc = col_ref[0, 0, i]
        gr[pl.ds(i, 1), :] = r4_ref[pl.ds(r, 1), :]
        gc[pl.ds(i, 1), :] = c4_ref[pl.ds(cc, 1), :]
        return c
    jax.lax.fori_loop(0, be, gather, 0, unroll=8)

    ea = ea_ref[...]
    ef = ef_ref[...]
    emb_pre = (jnp.dot(ea, e1e_ref[...], preferred_element_type=jnp.float32)
               + be1_ref[...] + gr[:, 0:128] + gc[:, 0:128])
    emb = (jnp.dot(jnp.maximum(emb_pre, 0.0), we2_ref[...],
                   preferred_element_type=jnp.float32) + be2_ref[...])
    gate_pre = (jnp.dot(ea, uattr_ref[...], preferred_element_type=jnp.float32)
                + jnp.dot(ef, uf_ref[...], preferred_element_type=jnp.float32)
                + jnp.dot(emb, uc_ref[...], preferred_element_type=jnp.float32)
                + bupd_ref[...] + gr[:, 128:256] + gc[:, 128:256])
    gate = jax.nn.sigmoid(gate_pre)
    nfe = gate * emb + (1.0 - gate) * ef
    newf_ref[...] = nfe

    ro = (jnp.dot(nfe, rf_ref[...], preferred_element_type=jnp.float32)
          + jnp.dot(ea, rattr_ref[...], preferred_element_type=jnp.float32)
          + gr[:, 256:257] + gc[:, 256:257])
    ro_ref[...] = ro
    gseg = gr[:, 257:258]
    gseg_ref[...] = gseg

    iota = jax.lax.broadcasted_iota(jnp.int32, (be, G), 1)
    mask = gseg.astype(jnp.int32) == iota
    cand = jnp.where(mask, ro, NEG)
    gm_ref[...] = jnp.maximum(gm_ref[...],
                              jnp.max(cand, axis=0, keepdims=True))


def _p4(rol2, col2, ea, ef, r4, c4, e1e, we2, be1, be2,
        uattr, uf, uc, bupd, rf, rattr, n, e):
    be = _blk(e, 2000)
    grid = e // be
    full = lambda i: (0, 0)
    return pl.pallas_call(
        _p4_body,
        grid=(grid,),
        in_specs=[
            pl.BlockSpec((1, 1, be), lambda i: (i, 0, 0), memory_space=pltpu.SMEM),
            pl.BlockSpec((1, 1, be), lambda i: (i, 0, 0), memory_space=pltpu.SMEM),
            pl.BlockSpec((be, A), lambda i: (i, 0)),
            pl.BlockSpec((be, H), lambda i: (i, 0)),
            pl.BlockSpec((n, 384), full),
            pl.BlockSpec((n, 384), full),
            pl.BlockSpec((A, H), full),
            pl.BlockSpec((H, H), full),
            pl.BlockSpec((1, H), full),
            pl.BlockSpec((1, H), full),
            pl.BlockSpec((A, H), full),
            pl.BlockSpec((H, H), full),
            pl.BlockSpec((H, H), full),
            pl.BlockSpec((1, H), full),
            pl.BlockSpec((H, 1), full),
            pl.BlockSpec((A, 1), full),
        ],
        out_specs=[
            pl.BlockSpec((be, H), lambda i: (i, 0)),
            pl.BlockSpec((be, 1), lambda i: (i, 0)),
            pl.BlockSpec((be, 1), lambda i: (i, 0)),
            pl.BlockSpec((1, G), full),
        ],
        out_shape=[
            jax.ShapeDtypeStruct((e, H), jnp.float32),
            jax.ShapeDtypeStruct((e, 1), jnp.float32),
            jax.ShapeDtypeStruct((e, 1), jnp.float32),
            jax.ShapeDtypeStruct((1, G), jnp.float32),
        ],
        scratch_shapes=[
            pltpu.VMEM((be, 384), jnp.float32),
            pltpu.VMEM((be, 384), jnp.float32),
        ],
        compiler_params=pltpu.CompilerParams(
            dimension_semantics=("arbitrary",)),
    )(rol2, col2, ea, ef, r4, c4, e1e, we2, be1, be2,
      uattr, uf, uc, bupd, rf, rattr)


def _p5_body(newf_ref, ro_ref, gseg_ref, gm_ref, ws_ref, bs_ref,
             conf_ref, gu, gs):
    @pl.when(pl.program_id(0) == 0)
    def _():
        gu[...] = jnp.zeros(gu.shape, jnp.float32)
        gs[...] = jnp.zeros(gs.shape, jnp.float32)

    be = ro_ref.shape[0]
    iota = jax.lax.broadcasted_iota(jnp.int32, (be, G), 1)
    mask = gseg_ref[...].astype(jnp.int32) == iota
    maskf = mask.astype(jnp.float32)
    gmrow = jnp.sum(jnp.where(mask, gm_ref[...], 0.0), axis=1, keepdims=True)
    ev = jnp.exp(ro_ref[...] - gmrow)
    dn = (((0,), (0,)), ((), ()))
    gs[...] = gs[...] + jax.lax.dot_general(
        maskf, ev, dn, preferred_element_type=jnp.float32)
    gu[...] = gu[...] + jax.lax.dot_general(
        maskf, ev * newf_ref[...], dn, preferred_element_type=jnp.float32)

    gf = gu[...] / (gs[...] + 1e-16)
    conf_ref[...] = jax.nn.sigmoid(
        jnp.dot(gf, ws_ref[...], preferred_element_type=jnp.float32)
        + bs_ref[...])


def _p5(newf, ro, gseg, gm, ws, bs, e):
    be = _blk(e, 2000)
    grid = e // be
    full = lambda i: (0, 0)
    return pl.pallas_call(
        _p5_body,
        grid=(grid,),
        in_specs=[
            pl.BlockSpec((be, H), lambda i: (i, 0)),
            pl.BlockSpec((be, 1), lambda i: (i, 0)),
            pl.BlockSpec((be, 1), lambda i: (i, 0)),
            pl.BlockSpec((1, G), full),
            pl.BlockSpec((H, 1), full),
            pl.BlockSpec((1, 1), full),
        ],
        out_specs=[pl.BlockSpec((G, 1), full)],
        out_shape=[jax.ShapeDtypeStruct((G, 1), jnp.float32)],
        scratch_shapes=[
            pltpu.VMEM((G, H), jnp.float32),
            pltpu.VMEM((G, 1), jnp.float32),
        ],
        compiler_params=pltpu.CompilerParams(
            dimension_semantics=("arbitrary",)),
    )(newf, ro, gseg, gm, ws, bs)


def kernel(x, hidden_edge_feat, edge_index, edge_attr, batch, num_graphs,
           W_agg_att, b_agg_att, W_e1, b_e1, W_e2, b_e2,
           W_upd, b_upd, W_ro_att, b_ro_att, W_s, b_s):
    n = x.shape[0]
    e = edge_index.shape[1]

    wa_r, wa_c = W_agg_att[0:D], W_agg_att[D:2 * D]
    wa_attr, wa_f = W_agg_att[2 * D:2 * D + A], W_agg_att[2 * D + A:]
    e1a, e1b = W_e1[0:D], W_e1[D:2 * D]
    e1c, e1d, e1e = W_e1[2 * D:3 * D], W_e1[3 * D:4 * D], W_e1[4 * D:]
    u_a, u_b = W_upd[0:D], W_upd[D:2 * D]
    u_attr = W_upd[2 * D:2 * D + A]
    u_f = W_upd[2 * D + A:2 * D + A + H]
    u_c = W_upd[2 * D + A + H:]
    r_a, r_b = W_ro_att[0:D], W_ro_att[D:2 * D]
    r_f, r_attr = W_ro_att[2 * D:2 * D + H], W_ro_att[2 * D + H:]

    wx = jnp.concatenate([wa_r, wa_c, r_a, r_b, e1a, e1c, u_a, u_b], axis=1)
    fbatch = batch.astype(jnp.float32).reshape(n, 1)
    bei = _blk(e, 2000)
    rol2 = edge_index[0].reshape(e // bei, 1, bei)
    col2 = edge_index[1].reshape(e // bei, 1, bei)
    be1 = (b_e1.reshape(1, H) + 0.0)
    be2 = b_e2.reshape(1, H)
    bupd = b_upd.reshape(1, H)
    bs = b_s.reshape(1, 1)
    del b_agg_att, b_ro_att, num_graphs

    scal, big = _k1(x, fbatch, wx, n)
    lg, m = _p1(rol2, col2, edge_attr, hidden_edge_feat, scal,
                wa_attr, wa_f, n, e)
    s, u = _p2(col2, lg, hidden_edge_feat, m, n, e)
    r4, c4 = _k3(u, s, big, scal, e1b, e1d, n)
    newf, ro, gseg, gm = _p4(rol2, col2, edge_attr, hidden_edge_feat,
                             r4, c4, e1e, W_e2, be1, be2,
                             u_attr, u_f, u_c, bupd, r_f, r_attr, n, e)
    (conf,) = _p5(newf, ro, gseg, gm, W_s, bs, e)
    return (newf, conf)


# speedup vs baseline: 5.0465x; 1.1969x over previous
import functools

import jax
import jax.numpy as jnp
from jax.experimental import pallas as pl
from jax.experimental.pallas import tpu as pltpu

D = 128
H = 128
A = 16
G = 64
NEG = -1e30


def _blk(n, pref):
    return pref if n % pref == 0 else n


def _k1_body(x_ref, fb_ref, wx_ref, scal_ref, big_ref):
    y = jnp.dot(x_ref[...], wx_ref[...], preferred_element_type=jnp.float32)
    bn = y.shape[0]
    scal_ref[...] = jnp.concatenate(
        [y[:, :4], fb_ref[...], jnp.zeros((bn, 3), jnp.float32)], axis=1)
    big_ref[...] = y[:, 4:]


def _k1(x, fbatch, wx, n):
    bn = _blk(n, 2000)
    grid = n // bn
    return pl.pallas_call(
        _k1_body,
        grid=(grid,),
        in_specs=[
            pl.BlockSpec((bn, D), lambda i: (i, 0)),
            pl.BlockSpec((bn, 1), lambda i: (i, 0)),
            pl.BlockSpec((D, 516), lambda i: (0, 0)),
        ],
        out_specs=[
            pl.BlockSpec((bn, 8), lambda i: (i, 0)),
            pl.BlockSpec((bn, 512), lambda i: (i, 0)),
        ],
        out_shape=[
            jax.ShapeDtypeStruct((n, 8), jnp.float32),
            jax.ShapeDtypeStruct((n, 512), jnp.float32),
        ],
        compiler_params=pltpu.CompilerParams(
            dimension_semantics=("arbitrary",)),
    )(x, fbatch, wx)


def _p1_body(rol_ref, col_ref, ea_ref, ef_ref, scal_ref, wattr_ref, wf_ref,
             lg_ref, m_ref, sgr, sgc):
    @pl.when(pl.program_id(0) == 0)
    def _():
        m_ref[...] = jnp.full(m_ref.shape, NEG, jnp.float32)

    be = ea_ref.shape[0]
    part = (jnp.dot(ea_ref[...], wattr_ref[...],
                    preferred_element_type=jnp.float32) +
            jnp.dot(ef_ref[...], wf_ref[...],
                    preferred_element_type=jnp.float32))

    def gather(i, c):
        r = rol_ref[0, 0, i]
        cc = col_ref[0, 0, i]
        sgr[pl.ds(i, 1), :] = scal_ref[pl.ds(r, 1), :]
        sgc[pl.ds(i, 1), :] = scal_ref[pl.ds(cc, 1), :]
        return c
    jax.lax.fori_loop(0, be, gather, 0, unroll=16)

    lg_ref[...] = part + sgr[:, 0:1] + sgc[:, 1:2]

    def smax(i, c):
        cc = col_ref[0, 0, i]
        lv = lg_ref[pl.ds(i, 1), :]
        m_ref[pl.ds(cc, 1), :] = jnp.maximum(m_ref[pl.ds(cc, 1), :], lv)
        return c
    jax.lax.fori_loop(0, be, smax, 0, unroll=16)


def _p1(rol2, col2, ea, ef, scal, wattr, wf, n, e):
    be = _blk(e, 2000)
    grid = e // be
    return pl.pallas_call(
        _p1_body,
        grid=(grid,),
        in_specs=[
            pl.BlockSpec((1, 1, be), lambda i: (i, 0, 0), memory_space=pltpu.SMEM),
            pl.BlockSpec((1, 1, be), lambda i: (i, 0, 0), memory_space=pltpu.SMEM),
            pl.BlockSpec((be, A), lambda i: (i, 0)),
            pl.BlockSpec((be, H), lambda i: (i, 0)),
            pl.BlockSpec((n, 8), lambda i: (0, 0)),
            pl.BlockSpec((A, 1), lambda i: (0, 0)),
            pl.BlockSpec((H, 1), lambda i: (0, 0)),
        ],
        out_specs=[
            pl.BlockSpec((be, 1), lambda i: (i, 0)),
            pl.BlockSpec((n, 1), lambda i: (0, 0)),
        ],
        out_shape=[
            jax.ShapeDtypeStruct((e, 1), jnp.float32),
            jax.ShapeDtypeStruct((n, 1), jnp.float32),
        ],
        scratch_shapes=[
            pltpu.VMEM((be, 8), jnp.float32),
            pltpu.VMEM((be, 8), jnp.float32),
        ],
        compiler_params=pltpu.CompilerParams(
            dimension_semantics=("arbitrary",)),
    )(rol2, col2, ea, ef, scal, wattr, wf)


def _p2_body(col_ref, lg_ref, ef_ref, m_ref, s_ref, u_ref):
    @pl.when(pl.program_id(0) == 0)
    def _():
        s_ref[...] = jnp.zeros(s_ref.shape, jnp.float32)
        u_ref[...] = jnp.zeros(u_ref.shape, jnp.float32)

    be = lg_ref.shape[0]

    def body(i, c):
        cc = col_ref[0, 0, i]
        ev = jnp.exp(lg_ref[pl.ds(i, 1), :] - m_ref[pl.ds(cc, 1), :])
        s_ref[pl.ds(cc, 1), :] = s_ref[pl.ds(cc, 1), :] + ev
        u_ref[pl.ds(cc, 1), :] = (u_ref[pl.ds(cc, 1), :] +
                                  ev * ef_ref[pl.ds(i, 1), :])
        return c
    jax.lax.fori_loop(0, be, body, 0, unroll=16)


def _p2(col2, lg, ef, m, n, e):
    be = _blk(e, 2000)
    grid = e // be
    return pl.pallas_call(
        _p2_body,
        grid=(grid,),
        in_specs=[
            pl.BlockSpec((1, 1, be), lambda i: (i, 0, 0), memory_space=pltpu.SMEM),
            pl.BlockSpec((be, 1), lambda i: (i, 0)),
            pl.BlockSpec((be, H), lambda i: (i, 0)),
            pl.BlockSpec((n, 1), lambda i: (0, 0)),
        ],
        out_specs=[
            pl.BlockSpec((n, 1), lambda i: (0, 0)),
            pl.BlockSpec((n, H), lambda i: (0, 0)),
        ],
        out_shape=[
            jax.ShapeDtypeStruct((n, 1), jnp.float32),
            jax.ShapeDtypeStruct((n, H), jnp.float32),
        ],
        compiler_params=pltpu.CompilerParams(
            dimension_semantics=("arbitrary",)),
    )(col2, lg, ef, m)


def _k3_body(u_ref, s_ref, big_ref, scal_ref, e1b_ref, e1d_ref,
             r4_ref, c4_ref):
    bn = u_ref.shape[0]
    nf = u_ref[...] / (s_ref[...] + 1e-16)
    t_r = big_ref[:, 0:128] + jnp.dot(nf, e1b_ref[...],
                                      preferred_element_type=jnp.float32)
    t_c = big_ref[:, 128:256] + jnp.dot(nf, e1d_ref[...],
                                        preferred_element_type=jnp.float32)
    z126 = jnp.zeros((bn, 126), jnp.float32)
    z127 = jnp.zeros((bn, 127), jnp.float32)
    r4_ref[...] = jnp.concatenate(
        [t_r, big_ref[:, 256:384],
         scal_ref[:, 2:3], scal_ref[:, 4:5], z126], axis=1)
    c4_ref[...] = jnp.concatenate(
        [t_c, big_ref[:, 384:512], scal_ref[:, 3:4], z127], axis=1)


def _k3(u, s, big, scal, e1b, e1d, n):
    bn = _blk(n, 2000)
    grid = n // bn
    return pl.pallas_call(
        _k3_body,
        grid=(grid,),
        in_specs=[
            pl.BlockSpec((bn, H), lambda i: (i, 0)),
            pl.BlockSpec((bn, 1), lambda i: (i, 0)),
            pl.BlockSpec((bn, 512), lambda i: (i, 0)),
            pl.BlockSpec((bn, 8), lambda i: (i, 0)),
            pl.BlockSpec((H, H), lambda i: (0, 0)),
            pl.BlockSpec((H, H), lambda i: (0, 0)),
        ],
        out_specs=[
            pl.BlockSpec((bn, 384), lambda i: (i, 0)),
            pl.BlockSpec((bn, 384), lambda i: (i, 0)),
        ],
        out_shape=[
            jax.ShapeDtypeStruct((n, 384), jnp.float32),
            jax.ShapeDtypeStruct((n, 384), jnp.float32),
        ],
        compiler_params=pltpu.CompilerParams(
            dimension_semantics=("arbitrary",)),
    )(u, s, big, scal, e1b, e1d)


def _p4_body(rol_ref, col_ref, ea_ref, ef_ref, r4_ref, c4_ref,
             e1e_ref, we2_ref, be1_ref, be2_ref,
             uattr_ref, uf_ref, uc_ref, bupd_ref, rf_ref, rattr_ref,
             newf_ref, ro_ref, gseg_ref, gm_ref, gr, gc):
    @pl.when(pl.program_id(0) == 0)
    def _():
        gm_ref[...] = jnp.full(gm_ref.shape, NEG, jnp.float32)

    be = ea_ref.shape[0]

    def gather(i, c):
        r = rol_ref[0, 0, i]
        cc = col_ref[0, 0, i]
        gr[pl.ds(i, 1), :] = r4_ref[pl.ds(r, 1), :]
        gc[pl.ds(i, 1), :] = c4_ref[pl.ds(cc, 1), :]
        return c
    jax.lax.fori_loop(0, be, gather, 0, unroll=16)

    ea = ea_ref[...]
    ef = ef_ref[...]
    emb_pre = (jnp.dot(ea, e1e_ref[...], preferred_element_type=jnp.float32)
               + be1_ref[...] + gr[:, 0:128] + gc[:, 0:128])
    emb = (jnp.dot(jnp.maximum(emb_pre, 0.0), we2_ref[...],
                   preferred_element_type=jnp.float32) + be2_ref[...])
    gate_pre = (jnp.dot(ea, uattr_ref[...], preferred_element_type=jnp.float32)
                + jnp.dot(ef, uf_ref[...], preferred_element_type=jnp.float32)
                + jnp.dot(emb, uc_ref[...], preferred_element_type=jnp.float32)
                + bupd_ref[...] + gr[:, 128:256] + gc[:, 128:256])
    gate = jax.nn.sigmoid(gate_pre)
    nfe = gate * emb + (1.0 - gate) * ef
    newf_ref[...] = nfe

    ro = (jnp.dot(nfe, rf_ref[...], preferred_element_type=jnp.float32)
          + jnp.dot(ea, rattr_ref[...], preferred_element_type=jnp.float32)
          + gr[:, 256:257] + gc[:, 256:257])
    ro_ref[...] = ro
    gseg = gr[:, 257:258]
    gseg_ref[...] = gseg

    iota = jax.lax.broadcasted_iota(jnp.int32, (be, G), 1)
    mask = gseg.astype(jnp.int32) == iota
    cand = jnp.where(mask, ro, NEG)
    gm_ref[...] = jnp.maximum(gm_ref[...],
                              jnp.max(cand, axis=0, keepdims=True))


def _p4(rol2, col2, ea, ef, r4, c4, e1e, we2, be1, be2,
        uattr, uf, uc, bupd, rf, rattr, n, e):
    be = _blk(e, 2000)
    grid = e // be
    full = lambda i: (0, 0)
    return pl.pallas_call(
        _p4_body,
        grid=(grid,),
        in_specs=[
            pl.BlockSpec((1, 1, be), lambda i: (i, 0, 0), memory_space=pltpu.SMEM),
            pl.BlockSpec((1, 1, be), lambda i: (i, 0, 0), memory_space=pltpu.SMEM),
            pl.BlockSpec((be, A), lambda i: (i, 0)),
            pl.BlockSpec((be, H), lambda i: (i, 0)),
            pl.BlockSpec((n, 384), full),
            pl.BlockSpec((n, 384), full),
            pl.BlockSpec((A, H), full),
            pl.BlockSpec((H, H), full),
            pl.BlockSpec((1, H), full),
            pl.BlockSpec((1, H), full),
            pl.BlockSpec((A, H), full),
            pl.BlockSpec((H, H), full),
            pl.BlockSpec((H, H), full),
            pl.BlockSpec((1, H), full),
            pl.BlockSpec((H, 1), full),
            pl.BlockSpec((A, 1), full),
        ],
        out_specs=[
            pl.BlockSpec((be, H), lambda i: (i, 0)),
            pl.BlockSpec((be, 1), lambda i: (i, 0)),
            pl.BlockSpec((be, 1), lambda i: (i, 0)),
            pl.BlockSpec((1, G), full),
        ],
        out_shape=[
            jax.ShapeDtypeStruct((e, H), jnp.float32),
            jax.ShapeDtypeStruct((e, 1), jnp.float32),
            jax.ShapeDtypeStruct((e, 1), jnp.float32),
            jax.ShapeDtypeStruct((1, G), jnp.float32),
        ],
        scratch_shapes=[
            pltpu.VMEM((be, 384), jnp.float32),
            pltpu.VMEM((be, 384), jnp.float32),
        ],
        compiler_params=pltpu.CompilerParams(
            dimension_semantics=("arbitrary",)),
    )(rol2, col2, ea, ef, r4, c4, e1e, we2, be1, be2,
      uattr, uf, uc, bupd, rf, rattr)


def _p5_body(newf_ref, ro_ref, gseg_ref, gm_ref, ws_ref, bs_ref,
             conf_ref, gu, gs):
    @pl.when(pl.program_id(0) == 0)
    def _():
        gu[...] = jnp.zeros(gu.shape, jnp.float32)
        gs[...] = jnp.zeros(gs.shape, jnp.float32)

    be = ro_ref.shape[0]
    iota = jax.lax.broadcasted_iota(jnp.int32, (be, G), 1)
    mask = gseg_ref[...].astype(jnp.int32) == iota
    maskf = mask.astype(jnp.float32)
    gmrow = jnp.sum(jnp.where(mask, gm_ref[...], 0.0), axis=1, keepdims=True)
    ev = jnp.exp(ro_ref[...] - gmrow)
    dn = (((0,), (0,)), ((), ()))
    gs[...] = gs[...] + jax.lax.dot_general(
        maskf, ev, dn, preferred_element_type=jnp.float32)
    gu[...] = gu[...] + jax.lax.dot_general(
        maskf, ev * newf_ref[...], dn, preferred_element_type=jnp.float32)

    gf = gu[...] / (gs[...] + 1e-16)
    conf_ref[...] = jax.nn.sigmoid(
        jnp.dot(gf, ws_ref[...], preferred_element_type=jnp.float32)
        + bs_ref[...])


def _p5(newf, ro, gseg, gm, ws, bs, e):
    be = _blk(e, 2000)
    grid = e // be
    full = lambda i: (0, 0)
    return pl.pallas_call(
        _p5_body,
        grid=(grid,),
        in_specs=[
            pl.BlockSpec((be, H), lambda i: (i, 0)),
            pl.BlockSpec((be, 1), lambda i: (i, 0)),
            pl.BlockSpec((be, 1), lambda i: (i, 0)),
            pl.BlockSpec((1, G), full),
            pl.BlockSpec((H, 1), full),
            pl.BlockSpec((1, 1), full),
        ],
        out_specs=[pl.BlockSpec((G, 1), full)],
        out_shape=[jax.ShapeDtypeStruct((G, 1), jnp.float32)],
        scratch_shapes=[
            pltpu.VMEM((G, H), jnp.float32),
            pltpu.VMEM((G, 1), jnp.float32),
        ],
        compiler_params=pltpu.CompilerParams(
            dimension_semantics=("arbitrary",)),
    )(newf, ro, gseg, gm, ws, bs)


def kernel(x, hidden_edge_feat, edge_index, edge_attr, batch, num_graphs,
           W_agg_att, b_agg_att, W_e1, b_e1, W_e2, b_e2,
           W_upd, b_upd, W_ro_att, b_ro_att, W_s, b_s):
    n = x.shape[0]
    e = edge_index.shape[1]

    wa_r, wa_c = W_agg_att[0:D], W_agg_att[D:2 * D]
    wa_attr, wa_f = W_agg_att[2 * D:2 * D + A], W_agg_att[2 * D + A:]
    e1a, e1b = W_e1[0:D], W_e1[D:2 * D]
    e1c, e1d, e1e = W_e1[2 * D:3 * D], W_e1[3 * D:4 * D], W_e1[4 * D:]
    u_a, u_b = W_upd[0:D], W_upd[D:2 * D]
    u_attr = W_upd[2 * D:2 * D + A]
    u_f = W_upd[2 * D + A:2 * D + A + H]
    u_c = W_upd[2 * D + A + H:]
    r_a, r_b = W_ro_att[0:D], W_ro_att[D:2 * D]
    r_f, r_attr = W_ro_att[2 * D:2 * D + H], W_ro_att[2 * D + H:]

    wx = jnp.concatenate([wa_r, wa_c, r_a, r_b, e1a, e1c, u_a, u_b], axis=1)
    fbatch = batch.astype(jnp.float32).reshape(n, 1)
    bei = _blk(e, 2000)
    rol2 = edge_index[0].reshape(e // bei, 1, bei)
    col2 = edge_index[1].reshape(e // bei, 1, bei)
    be1 = (b_e1.reshape(1, H) + 0.0)
    be2 = b_e2.reshape(1, H)
    bupd = b_upd.reshape(1, H)
    bs = b_s.reshape(1, 1)
    del b_agg_att, b_ro_att, num_graphs

    scal, big = _k1(x, fbatch, wx, n)
    lg, m = _p1(rol2, col2, edge_attr, hidden_edge_feat, scal,
                wa_attr, wa_f, n, e)
    s, u = _p2(col2, lg, hidden_edge_feat, m, n, e)
    r4, c4 = _k3(u, s, big, scal, e1b, e1d, n)
    newf, ro, gseg, gm = _p4(rol2, col2, edge_attr, hidden_edge_feat,
                             r4, c4, e1e, W_e2, be1, be2,
                             u_attr, u_f, u_c, bupd, r_f, r_attr, n, e)
    (conf,) = _p5(newf, ro, gseg, gm, W_s, bs, e)
    return (newf, conf)


# speedup vs baseline: 5.5823x; 1.1062x over previous
import functools

import jax
import jax.numpy as jnp
from jax.experimental import pallas as pl
from jax.experimental.pallas import tpu as pltpu

D = 128
H = 128
A = 16
G = 64
NEG = -1e30


def _blk(n, pref):
    return pref if n % pref == 0 else n


def _k1_body(x_ref, fb_ref, wx_ref, scal_ref, big_ref):
    y = jnp.dot(x_ref[...], wx_ref[...], preferred_element_type=jnp.float32)
    bn = y.shape[0]
    scal_ref[...] = jnp.concatenate(
        [y[:, :4], fb_ref[...], jnp.zeros((bn, 3), jnp.float32)], axis=1)
    big_ref[...] = y[:, 4:]


def _k1(x, fbatch, wx, n):
    bn = _blk(n, 2000)
    grid = n // bn
    return pl.pallas_call(
        _k1_body,
        grid=(grid,),
        in_specs=[
            pl.BlockSpec((bn, D), lambda i: (i, 0)),
            pl.BlockSpec((bn, 1), lambda i: (i, 0)),
            pl.BlockSpec((D, 516), lambda i: (0, 0)),
        ],
        out_specs=[
            pl.BlockSpec((bn, 8), lambda i: (i, 0)),
            pl.BlockSpec((bn, 512), lambda i: (i, 0)),
        ],
        out_shape=[
            jax.ShapeDtypeStruct((n, 8), jnp.float32),
            jax.ShapeDtypeStruct((n, 512), jnp.float32),
        ],
        compiler_params=pltpu.CompilerParams(
            dimension_semantics=("arbitrary",)),
    )(x, fbatch, wx)


def _p1_body(rol_ref, col_ref, ea_ref, ef_ref, scal_ref, wattr_ref, wf_ref,
             lg_ref, m_ref, sgr, sgc):
    @pl.when(pl.program_id(0) == 0)
    def _():
        m_ref[...] = jnp.full(m_ref.shape, NEG, jnp.float32)

    be = ea_ref.shape[0]
    part = (jnp.dot(ea_ref[...], wattr_ref[...],
                    preferred_element_type=jnp.float32) +
            jnp.dot(ef_ref[...], wf_ref[...],
                    preferred_element_type=jnp.float32))

    def gather(i, c):
        r = rol_ref[0, 0, i]
        cc = col_ref[0, 0, i]
        sgr[pl.ds(i, 1), :] = scal_ref[pl.ds(r, 1), :]
        sgc[pl.ds(i, 1), :] = scal_ref[pl.ds(cc, 1), :]
        return c
    jax.lax.fori_loop(0, be, gather, 0, unroll=32)

    lg_ref[...] = part + sgr[:, 0:1] + sgc[:, 1:2]

    def smax(i, c):
        cc = col_ref[0, 0, i]
        lv = lg_ref[pl.ds(i, 1), :]
        m_ref[pl.ds(cc, 1), :] = jnp.maximum(m_ref[pl.ds(cc, 1), :], lv)
        return c
    jax.lax.fori_loop(0, be, smax, 0, unroll=32)


def _p1(rol2, col2, ea, ef, scal, wattr, wf, n, e):
    be = _blk(e, 2000)
    grid = e // be
    return pl.pallas_call(
        _p1_body,
        grid=(grid,),
        in_specs=[
            pl.BlockSpec((1, 1, be), lambda i: (i, 0, 0), memory_space=pltpu.SMEM),
            pl.BlockSpec((1, 1, be), lambda i: (i, 0, 0), memory_space=pltpu.SMEM),
            pl.BlockSpec((be, A), lambda i: (i, 0)),
            pl.BlockSpec((be, H), lambda i: (i, 0)),
            pl.BlockSpec((n, 8), lambda i: (0, 0)),
            pl.BlockSpec((A, 1), lambda i: (0, 0)),
            pl.BlockSpec((H, 1), lambda i: (0, 0)),
        ],
        out_specs=[
            pl.BlockSpec((be, 1), lambda i: (i, 0)),
            pl.BlockSpec((n, 1), lambda i: (0, 0)),
        ],
        out_shape=[
            jax.ShapeDtypeStruct((e, 1), jnp.float32),
            jax.ShapeDtypeStruct((n, 1), jnp.float32),
        ],
        scratch_shapes=[
            pltpu.VMEM((be, 8), jnp.float32),
            pltpu.VMEM((be, 8), jnp.float32),
        ],
        compiler_params=pltpu.CompilerParams(
            dimension_semantics=("arbitrary",)),
    )(rol2, col2, ea, ef, scal, wattr, wf)


def _p2_body(col_ref, lg_ref, ef_ref, m_ref, s_ref, u_ref):
    @pl.when(pl.program_id(0) == 0)
    def _():
        s_ref[...] = jnp.zeros(s_ref.shape, jnp.float32)
        u_ref[...] = jnp.zeros(u_ref.shape, jnp.float32)

    be = lg_ref.shape[0]

    def body(i, c):
        cc = col_ref[0, 0, i]
        ev = jnp.exp(lg_ref[pl.ds(i, 1), :] - m_ref[pl.ds(cc, 1), :])
        s_ref[pl.ds(cc, 1), :] = s_ref[pl.ds(cc, 1), :] + ev
        u_ref[pl.ds(cc, 1), :] = (u_ref[pl.ds(cc, 1), :] +
                                  ev * ef_ref[pl.ds(i, 1), :])
        return c
    jax.lax.fori_loop(0, be, body, 0, unroll=32)


def _p2(col2, lg, ef, m, n, e):
    be = _blk(e, 2000)
    grid = e // be
    return pl.pallas_call(
        _p2_body,
        grid=(grid,),
        in_specs=[
            pl.BlockSpec((1, 1, be), lambda i: (i, 0, 0), memory_space=pltpu.SMEM),
            pl.BlockSpec((be, 1), lambda i: (i, 0)),
            pl.BlockSpec((be, H), lambda i: (i, 0)),
            pl.BlockSpec((n, 1), lambda i: (0, 0)),
        ],
        out_specs=[
            pl.BlockSpec((n, 1), lambda i: (0, 0)),
            pl.BlockSpec((n, H), lambda i: (0, 0)),
        ],
        out_shape=[
            jax.ShapeDtypeStruct((n, 1), jnp.float32),
            jax.ShapeDtypeStruct((n, H), jnp.float32),
        ],
        compiler_params=pltpu.CompilerParams(
            dimension_semantics=("arbitrary",)),
    )(col2, lg, ef, m)


def _k3_body(u_ref, s_ref, big_ref, scal_ref, e1b_ref, e1d_ref,
             r4_ref, c4_ref):
    bn = u_ref.shape[0]
    nf = u_ref[...] / (s_ref[...] + 1e-16)
    t_r = big_ref[:, 0:128] + jnp.dot(nf, e1b_ref[...],
                                      preferred_element_type=jnp.float32)
    t_c = big_ref[:, 128:256] + jnp.dot(nf, e1d_ref[...],
                                        preferred_element_type=jnp.float32)
    z126 = jnp.zeros((bn, 126), jnp.float32)
    z127 = jnp.zeros((bn, 127), jnp.float32)
    r4_ref[...] = jnp.concatenate(
        [t_r, big_ref[:, 256:384],
         scal_ref[:, 2:3], scal_ref[:, 4:5], z126], axis=1)
    c4_ref[...] = jnp.concatenate(
        [t_c, big_ref[:, 384:512], scal_ref[:, 3:4], z127], axis=1)


def _k3(u, s, big, scal, e1b, e1d, n):
    bn = _blk(n, 2000)
    grid = n // bn
    return pl.pallas_call(
        _k3_body,
        grid=(grid,),
        in_specs=[
            pl.BlockSpec((bn, H), lambda i: (i, 0)),
            pl.BlockSpec((bn, 1), lambda i: (i, 0)),
            pl.BlockSpec((bn, 512), lambda i: (i, 0)),
            pl.BlockSpec((bn, 8), lambda i: (i, 0)),
            pl.BlockSpec((H, H), lambda i: (0, 0)),
            pl.BlockSpec((H, H), lambda i: (0, 0)),
        ],
        out_specs=[
            pl.BlockSpec((bn, 384), lambda i: (i, 0)),
            pl.BlockSpec((bn, 384), lambda i: (i, 0)),
        ],
        out_shape=[
            jax.ShapeDtypeStruct((n, 384), jnp.float32),
            jax.ShapeDtypeStruct((n, 384), jnp.float32),
        ],
        compiler_params=pltpu.CompilerParams(
            dimension_semantics=("arbitrary",)),
    )(u, s, big, scal, e1b, e1d)


def _p4_body(rol_ref, col_ref, ea_ref, ef_ref, r4_ref, c4_ref,
             e1e_ref, we2_ref, be1_ref, be2_ref,
             uattr_ref, uf_ref, uc_ref, bupd_ref, rf_ref, rattr_ref,
             newf_ref, ro_ref, gseg_ref, gm_ref, gr, gc):
    @pl.when(pl.program_id(0) == 0)
    def _():
        gm_ref[...] = jnp.full(gm_ref.shape, NEG, jnp.float32)

    be = ea_ref.shape[0]

    def gather(i, c):
        r = rol_ref[0, 0, i]
        cc = col_ref[0, 0, i]
        gr[pl.ds(i, 1), :] = r4_ref[pl.ds(r, 1), :]
        gc[pl.ds(i, 1), :] = c4_ref[pl.ds(cc, 1), :]
        return c
    jax.lax.fori_loop(0, be, gather, 0, unroll=32)

    ea = ea_ref[...]
    ef = ef_ref[...]
    emb_pre = (jnp.dot(ea, e1e_ref[...], preferred_element_type=jnp.float32)
               + be1_ref[...] + gr[:, 0:128] + gc[:, 0:128])
    emb = (jnp.dot(jnp.maximum(emb_pre, 0.0), we2_ref[...],
                   preferred_element_type=jnp.float32) + be2_ref[...])
    gate_pre = (jnp.dot(ea, uattr_ref[...], preferred_element_type=jnp.float32)
                + jnp.dot(ef, uf_ref[...], preferred_element_type=jnp.float32)
                + jnp.dot(emb, uc_ref[...], preferred_element_type=jnp.float32)
                + bupd_ref[...] + gr[:, 128:256] + gc[:, 128:256])
    gate = jax.nn.sigmoid(gate_pre)
    nfe = gate * emb + (1.0 - gate) * ef
    newf_ref[...] = nfe

    ro = (jnp.dot(nfe, rf_ref[...], preferred_element_type=jnp.float32)
          + jnp.dot(ea, rattr_ref[...], preferred_element_type=jnp.float32)
          + gr[:, 256:257] + gc[:, 256:257])
    ro_ref[...] = ro
    gseg = gr[:, 257:258]
    gseg_ref[...] = gseg

    iota = jax.lax.broadcasted_iota(jnp.int32, (be, G), 1)
    mask = gseg.astype(jnp.int32) == iota
    cand = jnp.where(mask, ro, NEG)
    gm_ref[...] = jnp.maximum(gm_ref[...],
                              jnp.max(cand, axis=0, keepdims=True))


def _p4(rol2, col2, ea, ef, r4, c4, e1e, we2, be1, be2,
        uattr, uf, uc, bupd, rf, rattr, n, e):
    be = _blk(e, 2000)
    grid = e // be
    full = lambda i: (0, 0)
    return pl.pallas_call(
        _p4_body,
        grid=(grid,),
        in_specs=[
            pl.BlockSpec((1, 1, be), lambda i: (i, 0, 0), memory_space=pltpu.SMEM),
            pl.BlockSpec((1, 1, be), lambda i: (i, 0, 0), memory_space=pltpu.SMEM),
            pl.BlockSpec((be, A), lambda i: (i, 0)),
            pl.BlockSpec((be, H), lambda i: (i, 0)),
            pl.BlockSpec((n, 384), full),
            pl.BlockSpec((n, 384), full),
            pl.BlockSpec((A, H), full),
            pl.BlockSpec((H, H), full),
            pl.BlockSpec((1, H), full),
            pl.BlockSpec((1, H), full),
            pl.BlockSpec((A, H), full),
            pl.BlockSpec((H, H), full),
            pl.BlockSpec((H, H), full),
            pl.BlockSpec((1, H), full),
            pl.BlockSpec((H, 1), full),
            pl.BlockSpec((A, 1), full),
        ],
        out_specs=[
            pl.BlockSpec((be, H), lambda i: (i, 0)),
            pl.BlockSpec((be, 1), lambda i: (i, 0)),
            pl.BlockSpec((be, 1), lambda i: (i, 0)),
            pl.BlockSpec((1, G), full),
        ],
        out_shape=[
            jax.ShapeDtypeStruct((e, H), jnp.float32),
            jax.ShapeDtypeStruct((e, 1), jnp.float32),
            jax.ShapeDtypeStruct((e, 1), jnp.float32),
            jax.ShapeDtypeStruct((1, G), jnp.float32),
        ],
        scratch_shapes=[
            pltpu.VMEM((be, 384), jnp.float32),
            pltpu.VMEM((be, 384), jnp.float32),
        ],
        compiler_params=pltpu.CompilerParams(
            dimension_semantics=("arbitrary",)),
    )(rol2, col2, ea, ef, r4, c4, e1e, we2, be1, be2,
      uattr, uf, uc, bupd, rf, rattr)


def _p5_body(newf_ref, ro_ref, gseg_ref, gm_ref, ws_ref, bs_ref,
             conf_ref, gu, gs):
    @pl.when(pl.program_id(0) == 0)
    def _():
        gu[...] = jnp.zeros(gu.shape, jnp.float32)
        gs[...] = jnp.zeros(gs.shape, jnp.float32)

    be = ro_ref.shape[0]
    iota = jax.lax.broadcasted_iota(jnp.int32, (be, G), 1)
    mask = gseg_ref[...].astype(jnp.int32) == iota
    maskf = mask.astype(jnp.float32)
    gmrow = jnp.sum(jnp.where(mask, gm_ref[...], 0.0), axis=1, keepdims=True)
    ev = jnp.exp(ro_ref[...] - gmrow)
    dn = (((0,), (0,)), ((), ()))
    gs[...] = gs[...] + jax.lax.dot_general(
        maskf, ev, dn, preferred_element_type=jnp.float32)
    gu[...] = gu[...] + jax.lax.dot_general(
        maskf, ev * newf_ref[...], dn, preferred_element_type=jnp.float32)

    gf = gu[...] / (gs[...] + 1e-16)
    conf_ref[...] = jax.nn.sigmoid(
        jnp.dot(gf, ws_ref[...], preferred_element_type=jnp.float32)
        + bs_ref[...])


def _p5(newf, ro, gseg, gm, ws, bs, e):
    be = _blk(e, 2000)
    grid = e // be
    full = lambda i: (0, 0)
    return pl.pallas_call(
        _p5_body,
        grid=(grid,),
        in_specs=[
            pl.BlockSpec((be, H), lambda i: (i, 0)),
            pl.BlockSpec((be, 1), lambda i: (i, 0)),
            pl.BlockSpec((be, 1), lambda i: (i, 0)),
            pl.BlockSpec((1, G), full),
            pl.BlockSpec((H, 1), full),
            pl.BlockSpec((1, 1), full),
        ],
        out_specs=[pl.BlockSpec((G, 1), full)],
        out_shape=[jax.ShapeDtypeStruct((G, 1), jnp.float32)],
        scratch_shapes=[
            pltpu.VMEM((G, H), jnp.float32),
            pltpu.VMEM((G, 1), jnp.float32),
        ],
        compiler_params=pltpu.CompilerParams(
            dimension_semantics=("arbitrary",)),
    )(newf, ro, gseg, gm, ws, bs)


def kernel(x, hidden_edge_feat, edge_index, edge_attr, batch, num_graphs,
           W_agg_att, b_agg_att, W_e1, b_e1, W_e2, b_e2,
           W_upd, b_upd, W_ro_att, b_ro_att, W_s, b_s):
    n = x.shape[0]
    e = edge_index.shape[1]

    wa_r, wa_c = W_agg_att[0:D], W_agg_att[D:2 * D]
    wa_attr, wa_f = W_agg_att[2 * D:2 * D + A], W_agg_att[2 * D + A:]
    e1a, e1b = W_e1[0:D], W_e1[D:2 * D]
    e1c, e1d, e1e = W_e1[2 * D:3 * D], W_e1[3 * D:4 * D], W_e1[4 * D:]
    u_a, u_b = W_upd[0:D], W_upd[D:2 * D]
    u_attr = W_upd[2 * D:2 * D + A]
    u_f = W_upd[2 * D + A:2 * D + A + H]
    u_c = W_upd[2 * D + A + H:]
    r_a, r_b = W_ro_att[0:D], W_ro_att[D:2 * D]
    r_f, r_attr = W_ro_att[2 * D:2 * D + H], W_ro_att[2 * D + H:]

    wx = jnp.concatenate([wa_r, wa_c, r_a, r_b, e1a, e1c, u_a, u_b], axis=1)
    fbatch = batch.astype(jnp.float32).reshape(n, 1)
    bei = _blk(e, 2000)
    rol2 = edge_index[0].reshape(e // bei, 1, bei)
    col2 = edge_index[1].reshape(e // bei, 1, bei)
    be1 = (b_e1.reshape(1, H) + 0.0)
    be2 = b_e2.reshape(1, H)
    bupd = b_upd.reshape(1, H)
    bs = b_s.reshape(1, 1)
    del b_agg_att, b_ro_att, num_graphs

    scal, big = _k1(x, fbatch, wx, n)
    lg, m = _p1(rol2, col2, edge_attr, hidden_edge_feat, scal,
                wa_attr, wa_f, n, e)
    s, u = _p2(col2, lg, hidden_edge_feat, m, n, e)
    r4, c4 = _k3(u, s, big, scal, e1b, e1d, n)
    newf, ro, gseg, gm = _p4(rol2, col2, edge_attr, hidden_edge_feat,
                             r4, c4, e1e, W_e2, be1, be2,
                             u_attr, u_f, u_c, bupd, r_f, r_attr, n, e)
    (conf,) = _p5(newf, ro, gseg, gm, W_s, bs, e)
    return (newf, conf)
